# baseline (speedup 1.0000x reference)
import functools

import jax
import jax.numpy as jnp
from jax import lax
from jax.experimental import pallas as pl
from jax.experimental.pallas import tpu as pltpu
from jax.experimental.pallas import tpu_sc as plsc

B, A = 16384, 200
L = 16
NC, NS = 2, 16
NW = NC * NS
ROWS_PER_W = B // NW
CHUNK_ROWS = 64
GROUPS_PER_CHUNK = CHUNK_ROWS // L
NCHUNKS = ROWS_PER_W // CHUNK_ROWS
CHUNK_WORDS = CHUNK_ROWS * A


def _body(species_hbm, table_hbm, icpt_hbm, out_hbm,
          buf0, buf1, tab_v, icpt_v, out_v, sem0, sem1):
    wid = lax.axis_index("s") * NC + lax.axis_index("c")
    row0 = wid * ROWS_PER_W

    pltpu.sync_copy(table_hbm, tab_v)
    pltpu.sync_copy(icpt_hbm, icpt_v)
    tab = tab_v[...]
    icpt = icpt_v[...]

    bufs = (buf0, buf1)
    sems = (sem0, sem1)
    lane = lax.iota(jnp.int32, L)

    def start(c):
        off = (row0 + c * CHUNK_ROWS) * A
        return pltpu.async_copy(
            species_hbm.at[pl.ds(off, CHUNK_WORDS)], bufs[c % 2], sems[c % 2])

    copies = [start(0)]
    for c in range(NCHUNKS):
        if c + 1 < NCHUNKS:
            copies.append(start(c + 1))
        copies[c].wait()
        buf = bufs[c % 2]
        for g in range(GROUPS_PER_CHUNK):
            base = (g * L + lane) * A

            @plsc.parallel_loop(0, A, 1, unroll=8, carry=icpt)
            def acc_loop(a, acc, base=base, buf=buf):
                sv = plsc.load_gather(buf, [base + a])
                tv = lax.gather(
                    tab, sv[:, None],
                    lax.GatherDimensionNumbers(
                        offset_dims=(), collapsed_slice_dims=(0,),
                        start_index_map=(0,)),
                    slice_sizes=(1,),
                    mode=lax.GatherScatterMode.PROMISE_IN_BOUNDS)
                return acc + tv

            out_v[pl.ds((c * GROUPS_PER_CHUNK + g) * L, L)] = acc_loop

    pltpu.sync_copy(out_v, out_hbm.at[pl.ds(row0, ROWS_PER_W)])


_mesh = plsc.VectorSubcoreMesh(core_axis_name="c", subcore_axis_name="s",
                               num_cores=NC, num_subcores=NS)

_sc_call = pl.kernel(
    _body,
    out_type=jax.ShapeDtypeStruct((B,), jnp.float32),
    mesh=_mesh,
    scratch_types=[
        pltpu.VMEM((CHUNK_WORDS,), jnp.int32),
        pltpu.VMEM((CHUNK_WORDS,), jnp.int32),
        pltpu.VMEM((L,), jnp.float32),
        pltpu.VMEM((L,), jnp.float32),
        pltpu.VMEM((ROWS_PER_W,), jnp.float32),
        pltpu.SemaphoreType.DMA,
        pltpu.SemaphoreType.DMA,
    ],
    compiler_params=pltpu.CompilerParams(use_tc_tiling_on_sc=False,
                                         needs_layout_passes=False),
    name="energy_shifter_sc",
)


def kernel(species, energies, self_energies, intercept):
    tab16 = jnp.concatenate(
        [self_energies.astype(jnp.float32),
         jnp.zeros((L - self_energies.shape[0],), jnp.float32)])
    icpt16 = jnp.full((L,), intercept, jnp.float32)
    out = _sc_call(species.reshape(-1), tab16, icpt16)
    return (species, out)


# speedup vs baseline: 1.0106x; 1.0106x over previous
import functools

import jax
import jax.numpy as jnp
from jax import lax
from jax.experimental import pallas as pl
from jax.experimental.pallas import tpu as pltpu
from jax.experimental.pallas import tpu_sc as plsc

B, A = 16384, 200
L = 16
NC, NS = 2, 16
NW = NC * NS
ROWS_PER_W = B // NW
CHUNK_ROWS = 64
GROUPS_PER_CHUNK = CHUNK_ROWS // L
NCHUNKS = ROWS_PER_W // CHUNK_ROWS
CHUNK_WORDS = CHUNK_ROWS * A


def _body(species_hbm, table_hbm, icpt_hbm, out_hbm,
          buf0, buf1, tab_v, icpt_v, out_v, sem0, sem1):
    wid = lax.axis_index("s") * NC + lax.axis_index("c")
    row0 = wid * ROWS_PER_W

    pltpu.sync_copy(table_hbm, tab_v)
    pltpu.sync_copy(icpt_hbm, icpt_v)
    tab = tab_v[...]
    icpt = icpt_v[...]

    bufs = (buf0, buf1)
    sems = (sem0, sem1)
    lane = lax.iota(jnp.int32, L)

    def start(c):
        off = (row0 + c * CHUNK_ROWS) * A
        return pltpu.async_copy(
            species_hbm.at[pl.ds(off, CHUNK_WORDS)], bufs[c % 2], sems[c % 2])

    copies = [start(0)]
    for c in range(NCHUNKS):
        if c + 1 < NCHUNKS:
            copies.append(start(c + 1))
        copies[c].wait()
        buf = bufs[c % 2]
        for g in range(GROUPS_PER_CHUNK):
            base = (g * L + lane) * A

            zero = jnp.zeros((L,), jnp.float32)

            @plsc.parallel_loop(0, A, 1, unroll=25,
                                carry=(icpt, zero, zero, zero))
            def acc_loop(a, accs, base=base, buf=buf):
                a0, a1, a2, a3 = accs
                sv = plsc.load_gather(buf, [base + a])
                tv = lax.gather(
                    tab, sv[:, None],
                    lax.GatherDimensionNumbers(
                        offset_dims=(), collapsed_slice_dims=(0,),
                        start_index_map=(0,)),
                    slice_sizes=(1,),
                    mode=lax.GatherScatterMode.PROMISE_IN_BOUNDS)
                return (a1, a2, a3, a0 + tv)

            s0, s1, s2, s3 = acc_loop
            out_v[pl.ds((c * GROUPS_PER_CHUNK + g) * L, L)] = (
                (s0 + s1) + (s2 + s3))

    pltpu.sync_copy(out_v, out_hbm.at[pl.ds(row0, ROWS_PER_W)])


_mesh = plsc.VectorSubcoreMesh(core_axis_name="c", subcore_axis_name="s",
                               num_cores=NC, num_subcores=NS)

_sc_call = pl.kernel(
    _body,
    out_type=jax.ShapeDtypeStruct((B,), jnp.float32),
    mesh=_mesh,
    scratch_types=[
        pltpu.VMEM((CHUNK_WORDS,), jnp.int32),
        pltpu.VMEM((CHUNK_WORDS,), jnp.int32),
        pltpu.VMEM((L,), jnp.float32),
        pltpu.VMEM((L,), jnp.float32),
        pltpu.VMEM((ROWS_PER_W,), jnp.float32),
        pltpu.SemaphoreType.DMA,
        pltpu.SemaphoreType.DMA,
    ],
    compiler_params=pltpu.CompilerParams(use_tc_tiling_on_sc=False,
                                         needs_layout_passes=False),
    name="energy_shifter_sc",
)


def kernel(species, energies, self_energies, intercept):
    tab16 = jnp.concatenate(
        [self_energies.astype(jnp.float32),
         jnp.zeros((L - self_energies.shape[0],), jnp.float32)])
    icpt16 = jnp.full((L,), intercept, jnp.float32)
    out = _sc_call(species.reshape(-1), tab16, icpt16)
    return (species, out)
